# baseline (speedup 1.0000x reference)
import jax
import jax.numpy as jnp
import numpy as np
from jax import lax
from jax.experimental import pallas as pl
from jax.experimental.pallas import tpu as pltpu
from jax.experimental.pallas import tpu_sc as plsc

_NUSC_CLASS_FREQ = np.array(
    [944004, 1897170, 152386, 2391677, 16957802, 724139, 189027, 2074468,
     413451, 2384460, 5916653, 175883646, 4275424, 51393615, 61411620,
     105975596, 116424404, 1892500630], dtype=np.float64)
_CLASS_WEIGHTS = (1.0 / np.log(_NUSC_CLASS_FREQ + 0.001)).astype(np.float32)

C = 18
N = 640000
B = 512
NW = 32
VPT = N // NW
CHV = 1024
QN = N // CHV


def _sc_body(x_hbm, lab_hbm, hists_hbm, a_hbm, s_hbm,
             xbuf, labbuf, abuf, sbuf, hist, sem):
    wid = lax.axis_index("s") * 2 + lax.axis_index("c")
    iota = lax.broadcasted_iota(jnp.int32, (16,), 0)
    zeros_i = jnp.zeros((16,), jnp.int32)

    def zero_body(i, carry):
        hist[pl.ds(i * 16, 16)] = zeros_i
        return carry
    lax.fori_loop(0, (C * B) // 16, zero_body, 0)

    nchunk = (QN - wid + (NW - 1)) // NW

    def chunk_body(i, carry):
        q = wid + i * NW
        off_vox = q * CHV
        copies = [pltpu.make_async_copy(x_hbm.at[q, c],
                                        xbuf.at[pl.ds(c * CHV, CHV)], sem)
                  for c in range(C)]
        copies.append(pltpu.make_async_copy(
            lab_hbm.at[pl.ds(pl.multiple_of(off_vox, 8), CHV)], labbuf, sem))
        for cp in copies:
            cp.start()
        for cp in copies:
            cp.wait()

        def group_body(g, gcarry):
            v0 = g * 16
            lab16 = labbuf[pl.ds(v0, 16)]
            xs = [xbuf[pl.ds(c * CHV + v0, 16)] for c in range(C)]
            m = xs[0]
            for c in range(1, C):
                m = jnp.maximum(m, xs[c])
            es = [jnp.exp(xc - m) for xc in xs]
            s = es[0]
            for c in range(1, C):
                s = s + es[c]
            rinvb = float(B) / s
            for c in range(C):
                t = es[c] * rinvb
                fg = lab16 == c
                b = jnp.where(fg, float(B) - t, t).astype(jnp.int32)
                b = jnp.minimum(b, B - 1)
                val = jnp.where(fg, 65537, 1)
                plsc.addupdate_scatter(hist, [b + (c * B)], val)
            xlab = plsc.load_gather(xbuf, [lab16 * CHV + (v0 + iota)])
            abuf[pl.ds(v0, 16)] = xlab - m
            sbuf[pl.ds(v0, 16)] = s
            return gcarry
        lax.fori_loop(0, CHV // 16, group_body, 0)

        pltpu.sync_copy(abuf, a_hbm.at[pl.ds(pl.multiple_of(off_vox, 8), CHV)])
        pltpu.sync_copy(sbuf, s_hbm.at[pl.ds(pl.multiple_of(off_vox, 8), CHV)])
        return carry
    lax.fori_loop(0, nchunk, chunk_body, 0)

    pltpu.sync_copy(hist, hists_hbm.at[wid])


@jax.jit
def _sc_stage(x_flat, labels):
    mesh = plsc.VectorSubcoreMesh(core_axis_name="c", subcore_axis_name="s")
    f = pl.kernel(
        _sc_body,
        mesh=mesh,
        out_type=[
            jax.ShapeDtypeStruct((NW, C * B), jnp.int32),
            jax.ShapeDtypeStruct((N,), jnp.float32),
            jax.ShapeDtypeStruct((N,), jnp.float32),
        ],
        scratch_types=[
            pltpu.VMEM((CHV * C,), jnp.float32),
            pltpu.VMEM((CHV,), jnp.int32),
            pltpu.VMEM((CHV,), jnp.float32),
            pltpu.VMEM((CHV,), jnp.float32),
            pltpu.VMEM((C * B,), jnp.int32),
            pltpu.SemaphoreType.DMA,
        ],
        compiler_params=pltpu.CompilerParams(needs_layout_passes=False),
    )
    return f(x_flat, labels)


def _tc_body(h_ref, a_ref, s_ref, lab_ref, out_ref):
    hall = jnp.zeros((C, B), jnp.float32)
    hfg = jnp.zeros((C, B), jnp.float32)
    for w in range(NW):
        hw = h_ref[w]
        hall = hall + jnp.bitwise_and(hw, 0xFFFF).astype(jnp.float32)
        hfg = hfg + jnp.right_shift(hw, 16).astype(jnp.float32)

    ri = lax.broadcasted_iota(jnp.int32, (B, B), 0)
    ci = lax.broadcasted_iota(jnp.int32, (B, B), 1)
    lower = (ri >= ci).astype(jnp.float32)
    M = jax.lax.dot(hall, lower, precision=jax.lax.Precision.HIGHEST)
    F = jax.lax.dot(hfg, lower, precision=jax.lax.Precision.HIGHEST)
    G = F[:, 0:1]
    denom = jnp.maximum(M + G - F, 1.0)
    J = M / denom
    loss_c = (jnp.sum(J, axis=1, keepdims=True) - 0.5 * J[:, 0:1]) / float(B)
    present = (G > 0.0).astype(jnp.float32)
    lovasz = jnp.sum(loss_c * present) / jnp.maximum(jnp.sum(present), 1.0)

    lab = lab_ref[...]
    w = jnp.zeros(lab.shape, jnp.float32)
    for c in range(C):
        w = jnp.where(lab == c, float(_CLASS_WEIGHTS[c]), w)
    nll = jnp.log(s_ref[...]) - a_ref[...]
    ce = jnp.sum(w * nll) / jnp.sum(w)

    out_ref[...] = jnp.reshape(lovasz + ce, (1, 1))


@jax.jit
def _tc_stage(hists, a, s, labels):
    h2 = hists.reshape(NW, C, B)
    a2 = a.reshape(N // 1024, 1024)
    s2 = s.reshape(N // 1024, 1024)
    lab2 = labels.reshape(N // 1024, 1024)
    out = pl.pallas_call(
        _tc_body,
        out_shape=jax.ShapeDtypeStruct((1, 1), jnp.float32),
    )(h2, a2, s2, lab2)
    return out[0, 0]


def _tr_body(x_ref, o_ref):
    o_ref[...] = jnp.swapaxes(x_ref[...], 0, 1)[None]


@jax.jit
def _tc_transpose(seg_pred):
    return pl.pallas_call(
        _tr_body,
        grid=(QN,),
        in_specs=[pl.BlockSpec((CHV, C), lambda i: (i, 0))],
        out_specs=pl.BlockSpec((1, C, CHV), lambda i: (i, 0, 0)),
        out_shape=jax.ShapeDtypeStruct((QN, C, CHV), jnp.float32),
    )(seg_pred)


def kernel(seg_pred, voxel_semantics):
    labels = voxel_semantics.astype(jnp.int32)
    xt = _tc_transpose(seg_pred)
    hists, a, s = _sc_stage(xt, labels)
    return _tc_stage(hists, a, s, labels)


# speedup vs baseline: 1.2693x; 1.2693x over previous
import jax
import jax.numpy as jnp
import numpy as np
from jax import lax
from jax.experimental import pallas as pl
from jax.experimental.pallas import tpu as pltpu
from jax.experimental.pallas import tpu_sc as plsc

_NUSC_CLASS_FREQ = np.array(
    [944004, 1897170, 152386, 2391677, 16957802, 724139, 189027, 2074468,
     413451, 2384460, 5916653, 175883646, 4275424, 51393615, 61411620,
     105975596, 116424404, 1892500630], dtype=np.float64)
_CLASS_WEIGHTS = (1.0 / np.log(_NUSC_CLASS_FREQ + 0.001)).astype(np.float32)

C = 18
N = 640000
NH = N // 2
B = 512
NW = 32
VPT = NH // NW
CH = 2000
NCHUNK = VPT // CH
NG = CH // 16


def _sc_body(x_hbm, lab_hbm, hists_hbm, a_hbm, s_hbm,
             xbuf, labbuf, abuf, sbuf, hist):
    wid = lax.axis_index("s") * 2 + lax.axis_index("c")
    iota = lax.broadcasted_iota(jnp.int32, (16,), 0)
    iota18 = iota * C
    zeros_i = jnp.zeros((16,), jnp.int32)

    def zero_body(i, carry):
        hist[pl.ds(i * 16, 16)] = zeros_i
        return carry
    lax.fori_loop(0, (C * B) // 16, zero_body, 0)

    def chunk_body(ch, carry):
        off_vox = wid * VPT + ch * CH
        pltpu.sync_copy(x_hbm.at[pl.ds(pl.multiple_of(off_vox * C, 8), CH * C)],
                        xbuf)
        pltpu.sync_copy(lab_hbm.at[pl.ds(pl.multiple_of(off_vox, 8), CH)],
                        labbuf)

        def group_body(g, gcarry):
            base = g * (16 * C)
            lab16 = labbuf[pl.ds(g * 16, 16)]
            xs = [plsc.load_gather(xbuf, [iota18 + (base + c)])
                  for c in range(C)]
            m = xs[0]
            for c in range(1, C):
                m = jnp.maximum(m, xs[c])
            es = [jnp.exp(xc - m) for xc in xs]
            s = es[0]
            for c in range(1, C):
                s = s + es[c]
            rinvb = float(B) / s
            for c in range(C):
                t = es[c] * rinvb
                fg = lab16 == c
                b = jnp.where(fg, float(B) - t, t).astype(jnp.int32)
                b = jnp.minimum(b, B - 1)
                val = jnp.where(fg, 65537, 1)
                plsc.addupdate_scatter(hist, [b + (c * B)], val)
            xlab = plsc.load_gather(xbuf, [iota18 + base + lab16])
            abuf[pl.ds(g * 16, 16)] = xlab - m
            sbuf[pl.ds(g * 16, 16)] = s
            return gcarry
        lax.fori_loop(0, NG, group_body, 0)

        pltpu.sync_copy(abuf, a_hbm.at[pl.ds(pl.multiple_of(off_vox, 8), CH)])
        pltpu.sync_copy(sbuf, s_hbm.at[pl.ds(pl.multiple_of(off_vox, 8), CH)])
        return carry
    lax.fori_loop(0, NCHUNK, chunk_body, 0)

    pltpu.sync_copy(hist, hists_hbm.at[wid])


def _make_sc():
    mesh = plsc.VectorSubcoreMesh(core_axis_name="c", subcore_axis_name="s")
    return pl.kernel(
        _sc_body,
        mesh=mesh,
        out_type=[
            jax.ShapeDtypeStruct((NW, C * B), jnp.int32),
            jax.ShapeDtypeStruct((NH,), jnp.float32),
            jax.ShapeDtypeStruct((NH,), jnp.float32),
        ],
        scratch_types=[
            pltpu.VMEM((CH * C,), jnp.float32),
            pltpu.VMEM((CH,), jnp.int32),
            pltpu.VMEM((CH,), jnp.float32),
            pltpu.VMEM((CH,), jnp.float32),
            pltpu.VMEM((C * B,), jnp.int32),
        ],
        compiler_params=pltpu.CompilerParams(needs_layout_passes=False),
    )


def _tc_body(h0_ref, h1_ref, a0_ref, a1_ref, s0_ref, s1_ref,
             l0_ref, l1_ref, out_ref):
    hall = jnp.zeros((C, B), jnp.float32)
    hfg = jnp.zeros((C, B), jnp.float32)
    for href in (h0_ref, h1_ref):
        for w in range(NW):
            hw = href[w]
            hall = hall + jnp.bitwise_and(hw, 0xFFFF).astype(jnp.float32)
            hfg = hfg + jnp.right_shift(hw, 16).astype(jnp.float32)

    ri = lax.broadcasted_iota(jnp.int32, (B, B), 0)
    ci = lax.broadcasted_iota(jnp.int32, (B, B), 1)
    lower = (ri >= ci).astype(jnp.float32)
    M = jax.lax.dot(hall, lower, precision=jax.lax.Precision.HIGHEST)
    F = jax.lax.dot(hfg, lower, precision=jax.lax.Precision.HIGHEST)
    G = F[:, 0:1]
    denom = jnp.maximum(M + G - F, 1.0)
    J = M / denom
    loss_c = (jnp.sum(J, axis=1, keepdims=True) - 0.5 * J[:, 0:1]) / float(B)
    present = (G > 0.0).astype(jnp.float32)
    lovasz = jnp.sum(loss_c * present) / jnp.maximum(jnp.sum(present), 1.0)

    ce_num = 0.0
    ce_den = 0.0
    for a_ref, s_ref, lab_ref in ((a0_ref, s0_ref, l0_ref),
                                  (a1_ref, s1_ref, l1_ref)):
        lab = lab_ref[...]
        w = jnp.zeros(lab.shape, jnp.float32)
        for c in range(C):
            w = jnp.where(lab == c, float(_CLASS_WEIGHTS[c]), w)
        nll = jnp.log(s_ref[...]) - a_ref[...]
        ce_num = ce_num + jnp.sum(w * nll)
        ce_den = ce_den + jnp.sum(w)
    ce = ce_num / ce_den

    out_ref[...] = jnp.reshape(lovasz + ce, (1, 1))


@jax.jit
def _run(seg_pred, labels):
    sc = _make_sc()
    h0, a0, s0 = sc(seg_pred[:NH].reshape(-1), labels[:NH])
    h1, a1, s1 = sc(seg_pred[NH:].reshape(-1), labels[NH:])
    out = pl.pallas_call(
        _tc_body,
        out_shape=jax.ShapeDtypeStruct((1, 1), jnp.float32),
    )(h0.reshape(NW, C, B), h1.reshape(NW, C, B), a0.reshape(625, 512), a1.reshape(625, 512),
      s0.reshape(625, 512), s1.reshape(625, 512),
      labels[:NH].reshape(625, 512), labels[NH:].reshape(625, 512))
    return out[0, 0]


def kernel(seg_pred, voxel_semantics):
    return _run(seg_pred, voxel_semantics.astype(jnp.int32))


# speedup vs baseline: 1.4920x; 1.1755x over previous
import jax
import jax.numpy as jnp
import numpy as np
from jax import lax
from jax.experimental import pallas as pl
from jax.experimental.pallas import tpu as pltpu
from jax.experimental.pallas import tpu_sc as plsc

_NUSC_CLASS_FREQ = np.array(
    [944004, 1897170, 152386, 2391677, 16957802, 724139, 189027, 2074468,
     413451, 2384460, 5916653, 175883646, 4275424, 51393615, 61411620,
     105975596, 116424404, 1892500630], dtype=np.float64)
_CLASS_WEIGHTS = (1.0 / np.log(_NUSC_CLASS_FREQ + 0.001)).astype(np.float32)

C = 18
N = 640000
B = 512
NW = 32
VPT = N // NW
CH = 2000
NCHUNK = VPT // CH
NG = CH // 16


def _sc_body(x_hbm, lab_hbm, hists_hbm, a_hbm, s_hbm,
             xbuf, labbuf, abuf, sbuf, hist):
    wid = lax.axis_index("s") * 2 + lax.axis_index("c")
    iota = lax.broadcasted_iota(jnp.int32, (16,), 0)
    iota18 = iota * C
    zeros_i = jnp.zeros((16,), jnp.int32)

    def zero_body(i, carry):
        hist[pl.ds(i * 16, 16)] = zeros_i
        return carry
    lax.fori_loop(0, (C * B) // 16, zero_body, 0)

    def chunk_body(ch, carry):
        off_vox = wid * VPT + ch * CH
        pltpu.sync_copy(x_hbm.at[pl.ds(pl.multiple_of(off_vox * C, 8), CH * C)],
                        xbuf)
        pltpu.sync_copy(lab_hbm.at[pl.ds(pl.multiple_of(off_vox, 8), CH)],
                        labbuf)

        def group_body(g, gcarry):
            base = g * (16 * C)
            lab16 = labbuf[pl.ds(g * 16, 16)]
            xs = [plsc.load_gather(xbuf, [iota18 + (base + c)])
                  for c in range(C)]
            m = xs[0]
            for c in range(1, C):
                m = jnp.maximum(m, xs[c])
            es = [jnp.exp(xc - m) for xc in xs]
            s = es[0]
            for c in range(1, C):
                s = s + es[c]
            rinvb = float(B) / s
            for c in range(C):
                t = es[c] * rinvb
                fg = lab16 == c
                b = jnp.where(fg, float(B) - t, t).astype(jnp.int32)
                b = jnp.minimum(b, B - 1)
                val = jnp.where(fg, 65537, 1)
                plsc.addupdate_scatter(hist, [b + (c * B)], val)
            xlab = plsc.load_gather(xbuf, [iota18 + base + lab16])
            abuf[pl.ds(g * 16, 16)] = xlab - m
            sbuf[pl.ds(g * 16, 16)] = s
            return gcarry
        lax.fori_loop(0, NG, group_body, 0)

        pltpu.sync_copy(abuf, a_hbm.at[pl.ds(pl.multiple_of(off_vox, 8), CH)])
        pltpu.sync_copy(sbuf, s_hbm.at[pl.ds(pl.multiple_of(off_vox, 8), CH)])
        return carry
    lax.fori_loop(0, NCHUNK, chunk_body, 0)

    pltpu.sync_copy(hist, hists_hbm.at[wid])


@jax.jit
def _sc_stage(x_flat, labels):
    mesh = plsc.VectorSubcoreMesh(core_axis_name="c", subcore_axis_name="s")
    f = pl.kernel(
        _sc_body,
        mesh=mesh,
        out_type=[
            jax.ShapeDtypeStruct((NW, C * B), jnp.int32),
            jax.ShapeDtypeStruct((N,), jnp.float32),
            jax.ShapeDtypeStruct((N,), jnp.float32),
        ],
        scratch_types=[
            pltpu.VMEM((CH * C,), jnp.float32),
            pltpu.VMEM((CH,), jnp.int32),
            pltpu.VMEM((CH,), jnp.float32),
            pltpu.VMEM((CH,), jnp.float32),
            pltpu.VMEM((C * B,), jnp.int32),
        ],
        compiler_params=pltpu.CompilerParams(needs_layout_passes=False),
    )
    return f(x_flat, labels)


def _tc_body(h_ref, a_ref, s_ref, lab_ref, out_ref):
    hall = jnp.zeros((C, B), jnp.float32)
    hfg = jnp.zeros((C, B), jnp.float32)
    for w in range(NW):
        hw = h_ref[w]
        hall = hall + jnp.bitwise_and(hw, 0xFFFF).astype(jnp.float32)
        hfg = hfg + jnp.right_shift(hw, 16).astype(jnp.float32)

    ri = lax.broadcasted_iota(jnp.int32, (B, B), 0)
    ci = lax.broadcasted_iota(jnp.int32, (B, B), 1)
    lower = (ri >= ci).astype(jnp.float32)
    M = jax.lax.dot(hall, lower, precision=jax.lax.Precision.HIGHEST)
    F = jax.lax.dot(hfg, lower, precision=jax.lax.Precision.HIGHEST)
    G = F[:, 0:1]
    denom = jnp.maximum(M + G - F, 1.0)
    J = M / denom
    loss_c = (jnp.sum(J, axis=1, keepdims=True) - 0.5 * J[:, 0:1]) / float(B)
    present = (G > 0.0).astype(jnp.float32)
    lovasz = jnp.sum(loss_c * present) / jnp.maximum(jnp.sum(present), 1.0)

    lab = lab_ref[...]
    w = jnp.zeros(lab.shape, jnp.float32)
    for c in range(C):
        w = jnp.where(lab == c, float(_CLASS_WEIGHTS[c]), w)
    nll = jnp.log(s_ref[...]) - a_ref[...]
    ce = jnp.sum(w * nll) / jnp.sum(w)

    out_ref[...] = jnp.reshape(lovasz + ce, (1, 1))


@jax.jit
def _tc_stage(hists, a, s, labels):
    h2 = hists.reshape(NW, C, B)
    a2 = a.reshape(N // 1024, 1024)
    s2 = s.reshape(N // 1024, 1024)
    lab2 = labels.reshape(N // 1024, 1024)
    out = pl.pallas_call(
        _tc_body,
        out_shape=jax.ShapeDtypeStruct((1, 1), jnp.float32),
    )(h2, a2, s2, lab2)
    return out[0, 0]


def kernel(seg_pred, voxel_semantics):
    x_flat = seg_pred.reshape(-1)
    labels = voxel_semantics.astype(jnp.int32)
    hists, a, s = _sc_stage(x_flat, labels)
    return _tc_stage(hists, a, s, labels)
